# baseline (speedup 1.0000x reference)
import jax
import jax.numpy as jnp
from jax import lax
from jax.experimental import pallas as pl
from jax.experimental.pallas import tpu as pltpu
from jax.experimental.pallas import tpu_sc as plsc

F = 16
NC = 2
NS = 16
NW = NC * NS


def kernel(user, item, metadata, user_table, item_table, meta_table0,
           meta_table1, lin_user, lin_item, lin_meta0, lin_meta1):
    b = user.shape[0]
    assert b % (8 * NW) == 0
    bpw = b // NW

    m0c = metadata[:, 0].astype(jnp.int32)
    m1c = metadata[:, 1].astype(jnp.int32)
    user = user.astype(jnp.int32)
    item = item.astype(jnp.int32)
    lu_flat = lin_user.reshape(-1)
    li_flat = lin_item.reshape(-1)
    l0_flat = lin_meta0.reshape(-1)
    l1_flat = lin_meta1.reshape(-1)

    mesh = plsc.VectorSubcoreMesh(
        core_axis_name="c", subcore_axis_name="s",
        num_cores=NC, num_subcores=NS)

    def body(user_hbm, item_hbm, m0_hbm, m1_hbm,
             ut_hbm, it_hbm, t0_hbm, t1_hbm,
             lu_hbm, li_hbm, l0_hbm, l1_hbm,
             out_hbm,
             uidx, iidx, m0idx, m1idx,
             urows, irows, arows, brows,
             lu, li, l0, l1,
             pwbuf, outbuf, sem):
        wid = lax.axis_index("s") * NC + lax.axis_index("c")
        base = wid * bpw

        pltpu.sync_copy(user_hbm.at[pl.ds(base, bpw)], uidx)
        pltpu.sync_copy(item_hbm.at[pl.ds(base, bpw)], iidx)
        pltpu.sync_copy(m0_hbm.at[pl.ds(base, bpw)], m0idx)
        pltpu.sync_copy(m1_hbm.at[pl.ds(base, bpw)], m1idx)

        cps = [
            pltpu.async_copy(ut_hbm.at[uidx], urows, sem),
            pltpu.async_copy(it_hbm.at[iidx], irows, sem),
            pltpu.async_copy(t0_hbm.at[m0idx], arows, sem),
            pltpu.async_copy(t1_hbm.at[m1idx], brows, sem),
            pltpu.async_copy(lu_hbm.at[uidx], lu, sem),
            pltpu.async_copy(li_hbm.at[iidx], li, sem),
            pltpu.async_copy(l0_hbm.at[m0idx], l0, sem),
            pltpu.async_copy(l1_hbm.at[m1idx], l1, sem),
        ]
        for cp in cps:
            cp.wait()

        last_lane = lax.iota(jnp.int32, F) == (F - 1)

        @plsc.parallel_loop(0, bpw, unroll=8)
        def _(j):
            u = urows[j]
            it = irows[j]
            a = arows[j]
            c = brows[j]
            s = u + it + a + c
            q = s * s - u * u - it * it - a * a - c * c
            cs = plsc.cumsum(q)
            plsc.store_scatter(pwbuf, [jnp.full((F,), j, jnp.int32)], cs,
                               mask=last_lane)

        for g in range(bpw // F):
            sl = pl.ds(g * F, F)
            x = lu[sl] + li[sl] + l0[sl] + l1[sl] + 0.5 * pwbuf[sl]
            outbuf[sl] = 1.0 / (1.0 + jnp.exp(-x))

        pltpu.sync_copy(outbuf, out_hbm.at[pl.ds(base, bpw)])

    fm = pl.kernel(
        body,
        out_type=jax.ShapeDtypeStruct((b,), jnp.float32),
        mesh=mesh,
        compiler_params=pltpu.CompilerParams(
            needs_layout_passes=False, use_tc_tiling_on_sc=False),
        scratch_types=[
            pltpu.VMEM((bpw,), jnp.int32),
            pltpu.VMEM((bpw,), jnp.int32),
            pltpu.VMEM((bpw,), jnp.int32),
            pltpu.VMEM((bpw,), jnp.int32),
            pltpu.VMEM((bpw, F), jnp.float32),
            pltpu.VMEM((bpw, F), jnp.float32),
            pltpu.VMEM((bpw, F), jnp.float32),
            pltpu.VMEM((bpw, F), jnp.float32),
            pltpu.VMEM((bpw,), jnp.float32),
            pltpu.VMEM((bpw,), jnp.float32),
            pltpu.VMEM((bpw,), jnp.float32),
            pltpu.VMEM((bpw,), jnp.float32),
            pltpu.VMEM((bpw,), jnp.float32),
            pltpu.VMEM((bpw,), jnp.float32),
            pltpu.SemaphoreType.DMA,
        ],
    )
    return fm(user, item, m0c, m1c,
              user_table, item_table, meta_table0, meta_table1,
              lu_flat, li_flat, l0_flat, l1_flat)
